# baseline (speedup 1.0000x reference)
import functools

import jax
import jax.numpy as jnp
from jax import lax
from jax.experimental import pallas as pl
from jax.experimental.pallas import tpu as pltpu
from jax.experimental.pallas import tpu_sc as plsc

_NN = 50000
_E = 128
_U = 32
_L = 4
_B = 4096
_NS = 10

_NW = 32
_RPW = _B // _NW
_BB = 8
_NBLK = _RPW // _BB
_GPB = _BB * _L * _NS
_GCS = (128, 128, 64)
_IDXN = _RPW * _L * _NS


def _sc_body(neigh_hbm, offs_hbm, nte_hbm,
             s_hbm,
             neigh_v, offs_v, buf0_v, buf1_v, sbuf_v,
             sem0, sem1):
    c = lax.axis_index("c")
    s = lax.axis_index("s")
    wid = s * 2 + c
    row0 = wid * _RPW

    pltpu.sync_copy(neigh_hbm.at[pl.ds(row0 * _L * _NS, _IDXN)], neigh_v)
    pltpu.sync_copy(offs_hbm, offs_v)

    def idx_body(i, carry):
        ofs = offs_v[pl.ds(lax.rem(i, 5) * 16, 16)]
        nv = neigh_v[pl.ds(i * 16, 16)]
        neigh_v[pl.ds(i * 16, 16)] = nv * _L + ofs
        return carry

    lax.fori_loop(0, _IDXN // 16, idx_body, 0)

    bufs = (buf0_v, buf1_v)
    sems = (sem0, sem1)

    def fire(blk):
        gbase = blk * _GPB
        buf = bufs[blk % 2]
        sem = sems[blk % 2]
        cps = []
        off = 0
        for gc in _GCS:
            cps.append(pltpu.async_copy(
                nte_hbm.at[neigh_v.at[pl.ds(gbase + off, gc)]],
                buf.at[pl.ds(off, gc)],
                sem,
            ))
            off += gc
        return cps

    pend = fire(0)
    for blk in range(_NBLK):
        nxt = fire(blk + 1) if blk + 1 < _NBLK else []
        for cp in pend:
            cp.wait()
        pend = nxt
        buf = bufs[blk % 2]

        def acc_body(d, carry):
            r0 = d * _NS
            l = d % _L
            col = l * _U
            a0 = buf[r0, pl.ds(0, 16)]
            a1 = buf[r0, pl.ds(16, 16)]
            for r in range(1, _NS):
                a0 = a0 + buf[r0 + r, pl.ds(0, 16)]
                a1 = a1 + buf[r0 + r, pl.ds(16, 16)]
            b_loc = d // _L
            sbuf_v[blk * _BB + b_loc, pl.ds(col, 16)] = a0
            sbuf_v[blk * _BB + b_loc, pl.ds(col + 16, 16)] = a1
            return carry

        lax.fori_loop(0, _BB * _L, acc_body, 0)

    pltpu.sync_copy(sbuf_v, s_hbm.at[pl.ds(row0, _RPW)])


def _sc_stage(neigh_flat, offs, nte_flat):
    f = pl.kernel(
        _sc_body,
        out_type=jax.ShapeDtypeStruct((_B, _E), jnp.float32),
        mesh=plsc.VectorSubcoreMesh(core_axis_name="c", subcore_axis_name="s"),
        scratch_types=[
            pltpu.VMEM((_IDXN,), jnp.int32),
            pltpu.VMEM((80,), jnp.int32),
            pltpu.VMEM((_GPB, _U), jnp.float32),
            pltpu.VMEM((_GPB, _U), jnp.float32),
            pltpu.VMEM((_RPW, _E), jnp.float32),
            pltpu.SemaphoreType.DMA,
            pltpu.SemaphoreType.DMA,
        ],
        compiler_params=pltpu.CompilerParams(use_tc_tiling_on_sc=False),
    )
    return f(neigh_flat, offs, nte_flat)


def _base_body(ti_hbm, nemb_hbm, base_hbm, ti_v, bbuf_v, sem):
    wid = lax.axis_index("s") * 2 + lax.axis_index("c")
    row0 = wid * _RPW
    pltpu.sync_copy(ti_hbm.at[pl.ds(row0, _RPW)], ti_v)
    pltpu.async_copy(nemb_hbm.at[ti_v], bbuf_v, sem).wait()
    pltpu.sync_copy(bbuf_v, base_hbm.at[pl.ds(row0, _RPW)])


def _base_stage(ti, nemb):
    f = pl.kernel(
        _base_body,
        out_type=jax.ShapeDtypeStruct((_B, _E), jnp.float32),
        mesh=plsc.VectorSubcoreMesh(core_axis_name="c", subcore_axis_name="s"),
        scratch_types=[
            pltpu.VMEM((_RPW,), jnp.int32),
            pltpu.VMEM((_RPW, _E), jnp.float32),
            pltpu.SemaphoreType.DMA,
        ],
    )
    return f(ti, nemb)


_TB = 256
_NTB = _B // _TB


def _tc_body(ti_col_ref, ti_row_ref, tt_ref, shi_ref, slo_ref, base_ref,
             w1c_ref, w2c_ref, wc_ref, o_ref):
    tcol = ti_col_ref[...]
    trow = ti_row_ref[...]
    eq = (tcol == trow).astype(jnp.bfloat16)
    acc = (jnp.dot(eq, shi_ref[...], preferred_element_type=jnp.float32)
           + jnp.dot(eq, slo_ref[...], preferred_element_type=jnp.float32))

    tt = tt_ref[...]
    w2c = w2c_ref[...]
    nte_l = [acc[:, l * _U:(l + 1) * _U] for l in range(_L)]
    att_s = []
    for l in range(_L):
        h = jnp.dot(nte_l[l], w1c_ref[...],
                    preferred_element_type=jnp.float32)
        g = jnp.tanh(h) * w2c
        sc = jnp.zeros((_TB, 1), jnp.float32)
        for t in range(_L):
            cs = jnp.sum(g[:, t * _U:(t + 1) * _U], axis=1, keepdims=True)
            sc = jnp.where(tt == t, cs, sc)
        att_s.append(sc)
    m = jnp.maximum(jnp.maximum(att_s[0], att_s[1]),
                    jnp.maximum(att_s[2], att_s[3]))
    e = [jnp.exp(x - m) for x in att_s]
    z = e[0] + e[1] + e[2] + e[3]
    nta = ((e[0] / z) * nte_l[0] + (e[1] / z) * nte_l[1]
           + (e[2] / z) * nte_l[2] + (e[3] / z) * nte_l[3])
    z_all = jnp.dot(nta, wc_ref[...],
                    preferred_element_type=jnp.float32)
    proj = jnp.zeros((_TB, _E), jnp.float32)
    for t in range(_L):
        proj = jnp.where(tt == t, z_all[:, t * _E:(t + 1) * _E], proj)
    pre = base_ref[...] + proj
    nrm = jnp.sqrt(jnp.sum(pre * pre, axis=1, keepdims=True))
    o_ref[...] = pre / jnp.maximum(nrm, 1e-12)


def _tc_stage(ti_col, ti_row, tt_col, s_hi, s_lo, base, w1c, w2c, wc):
    return pl.pallas_call(
        _tc_body,
        grid=(_NTB,),
        in_specs=[
            pl.BlockSpec((_TB, 1), lambda i: (i, 0)),
            pl.BlockSpec((1, _B), lambda i: (0, 0)),
            pl.BlockSpec((_TB, 1), lambda i: (i, 0)),
            pl.BlockSpec((_B, _E), lambda i: (0, 0)),
            pl.BlockSpec((_B, _E), lambda i: (0, 0)),
            pl.BlockSpec((_TB, _E), lambda i: (i, 0)),
            pl.BlockSpec((_U, _L * _U), lambda i: (0, 0)),
            pl.BlockSpec((1, _L * _U), lambda i: (0, 0)),
            pl.BlockSpec((_U, _L * _E), lambda i: (0, 0)),
        ],
        out_specs=pl.BlockSpec((_TB, _E), lambda i: (i, 0)),
        out_shape=jax.ShapeDtypeStruct((_B, _E), jnp.float32),
        compiler_params=pltpu.CompilerParams(
            dimension_semantics=("arbitrary",),
        ),
    )(ti_col, ti_row, tt_col, s_hi, s_lo, base, w1c, w2c, wc)


def kernel(train_inputs, train_types, node_neigh, node_embeddings,
           node_type_embeddings, trans_weights, trans_weights_s1,
           trans_weights_s2):
    ti = train_inputs.astype(jnp.int32)
    tt = train_types.astype(jnp.int32)
    neigh_flat = node_neigh.astype(jnp.int32).reshape(-1)
    nte_flat = node_type_embeddings.reshape(_NN * _L, _U)
    offs = jnp.tile(jnp.repeat(jnp.arange(_L, dtype=jnp.int32), _NS), 2)
    base = _base_stage(ti, node_embeddings)
    s = _sc_stage(neigh_flat, offs, nte_flat)
    s_hi = s.astype(jnp.bfloat16)
    s_lo = (s - s_hi.astype(jnp.float32)).astype(jnp.bfloat16)
    w1c = jnp.concatenate([trans_weights_s1[t] for t in range(_L)], axis=1)
    w2c = trans_weights_s2[:, :, 0].reshape(1, _L * _U)
    wc = jnp.concatenate([trans_weights[t] for t in range(_L)], axis=1)
    return _tc_stage(ti.reshape(_B, 1), ti.reshape(1, _B), tt.reshape(_B, 1),
                     s_hi, s_lo, base, w1c, w2c, wc)


# speedup vs baseline: 1.2022x; 1.2022x over previous
import functools

import jax
import jax.numpy as jnp
from jax import lax
from jax.experimental import pallas as pl
from jax.experimental.pallas import tpu as pltpu
from jax.experimental.pallas import tpu_sc as plsc

_NN = 50000
_E = 128
_U = 32
_L = 4
_B = 4096
_NS = 10

_NW = 32
_RPW = _B // _NW
_BB = 8
_NBLK = _RPW // _BB
_GPB = _BB * _L * _NS
_GCS = (128, 128, 64)
_IDXN = _RPW * _L * _NS


def _sc_body(neigh_hbm, ti_hbm, nte_hbm, nemb_hbm,
             s_hbm, base_hbm, owner_hbm,
             neigh_v, buf0_v, buf1_v, sbuf_v, ti_v, bbuf_v, rid_v,
             sem0, sem1, bsem, osem):
    c = lax.axis_index("c")
    s = lax.axis_index("s")
    wid = s * 2 + c
    row0 = wid * _RPW

    pltpu.sync_copy(neigh_hbm.at[pl.ds(row0 * _L * _NS, _IDXN)], neigh_v)
    pltpu.sync_copy(ti_hbm.at[pl.ds(row0, _RPW)], ti_v)
    base_cp = pltpu.async_copy(nemb_hbm.at[ti_v], bbuf_v, bsem)

    for j in range(_RPW // 16):
        rid_v[pl.ds(j * 16, 16)] = lax.iota(jnp.int32, 16) + (row0 + j * 16)
    pltpu.async_copy(rid_v, owner_hbm.at[ti_v], osem).wait()

    bufs = (buf0_v, buf1_v)
    sems = (sem0, sem1)

    def fire(blk):
        gbase = blk * _GPB
        buf = bufs[blk % 2]
        sem = sems[blk % 2]
        cps = []
        off = 0
        for gc in _GCS:
            cps.append(pltpu.async_copy(
                nte_hbm.at[neigh_v.at[pl.ds(gbase + off, gc)]],
                buf.at[pl.ds(off, gc)],
                sem,
            ))
            off += gc
        return cps

    pend = fire(0)
    for blk in range(_NBLK):
        nxt = fire(blk + 1) if blk + 1 < _NBLK else []
        for cp in pend:
            cp.wait()
        pend = nxt
        buf = bufs[blk % 2]

        def acc_body(d, carry):
            r0 = d * _NS
            l = d % _L
            col = l * _U
            a0 = buf[r0, pl.ds(col, 16)]
            a1 = buf[r0, pl.ds(col + 16, 16)]
            for r in range(1, _NS):
                a0 = a0 + buf[r0 + r, pl.ds(col, 16)]
                a1 = a1 + buf[r0 + r, pl.ds(col + 16, 16)]
            b_loc = d // _L
            sbuf_v[blk * _BB + b_loc, pl.ds(col, 16)] = a0
            sbuf_v[blk * _BB + b_loc, pl.ds(col + 16, 16)] = a1
            return carry

        lax.fori_loop(0, _BB * _L, acc_body, 0)

    base_cp.wait()
    pltpu.sync_copy(sbuf_v, s_hbm.at[pl.ds(row0, _RPW)])
    pltpu.sync_copy(bbuf_v, base_hbm.at[pl.ds(row0, _RPW)])


def _sc_stage(neigh_flat, ti, nte_flat, nemb):
    f = pl.kernel(
        _sc_body,
        out_type=(
            jax.ShapeDtypeStruct((_B, _E), jnp.float32),
            jax.ShapeDtypeStruct((_B, _E), jnp.float32),
            jax.ShapeDtypeStruct((_NN,), jnp.int32),
        ),
        mesh=plsc.VectorSubcoreMesh(core_axis_name="c", subcore_axis_name="s"),
        scratch_types=[
            pltpu.VMEM((_IDXN,), jnp.int32),
            pltpu.VMEM((_GPB, _L * _U), jnp.float32),
            pltpu.VMEM((_GPB, _L * _U), jnp.float32),
            pltpu.VMEM((_RPW, _E), jnp.float32),
            pltpu.VMEM((_RPW,), jnp.int32),
            pltpu.VMEM((_RPW, _E), jnp.float32),
            pltpu.VMEM((_RPW,), jnp.int32),
            pltpu.SemaphoreType.DMA,
            pltpu.SemaphoreType.DMA,
            pltpu.SemaphoreType.DMA,
            pltpu.SemaphoreType.DMA,
        ],
    )
    return f(neigh_flat, ti, nte_flat, nemb)


_NSUB = 16
_TPS = _B // _NSUB


def _dedup_body(ti_hbm, s_hbm, owner_hbm, zeros_hbm, part_hbm,
                table_sh, ti_v, rep_v, sbuf_v, ti2_v, rep2_v, obuf_v, sem):
    c = lax.axis_index("c")
    s = lax.axis_index("s")
    wid = s * 2 + c
    row0 = wid * _RPW
    pltpu.sync_copy(ti_hbm.at[pl.ds(row0, _RPW)], ti_v)
    pltpu.async_copy(owner_hbm.at[ti_v], rep_v, sem).wait()
    pltpu.sync_copy(s_hbm.at[pl.ds(row0, _RPW)], sbuf_v)
    pltpu.sync_copy(zeros_hbm, table_sh.at[pl.ds(s * _TPS, _TPS)])
    plsc.subcore_barrier()
    pltpu.sync_copy(sbuf_v, table_sh.at[rep_v], add=True)
    plsc.subcore_barrier()
    g0 = s * _TPS
    pltpu.sync_copy(ti_hbm.at[pl.ds(g0, _TPS)], ti2_v)
    cps = [pltpu.async_copy(owner_hbm.at[ti2_v.at[pl.ds(k * 128, 128)]],
                            rep2_v.at[pl.ds(k * 128, 128)], sem)
           for k in range(_TPS // 128)]
    for cp in cps:
        cp.wait()
    cps = [pltpu.async_copy(table_sh.at[rep2_v.at[pl.ds(k * 128, 128)]],
                            obuf_v.at[pl.ds(k * 128, 128)], sem)
           for k in range(_TPS // 128)]
    for cp in cps:
        cp.wait()
    pltpu.sync_copy(obuf_v, part_hbm.at[pl.ds(c * _B + g0, _TPS)])


def _dedup_stage(ti, s, owner, zeros):
    f = pl.kernel(
        _dedup_body,
        out_type=jax.ShapeDtypeStruct((2 * _B, _E), jnp.float32),
        mesh=plsc.VectorSubcoreMesh(core_axis_name="c", subcore_axis_name="s"),
        scratch_types=[
            pltpu.VMEM_SHARED((_B, _E), jnp.float32),
            pltpu.VMEM((_RPW,), jnp.int32),
            pltpu.VMEM((_RPW,), jnp.int32),
            pltpu.VMEM((_RPW, _E), jnp.float32),
            pltpu.VMEM((_TPS,), jnp.int32),
            pltpu.VMEM((_TPS,), jnp.int32),
            pltpu.VMEM((_TPS, _E), jnp.float32),
            pltpu.SemaphoreType.DMA,
        ],
    )
    return f(ti, s, owner, zeros)


_TB = 256
_NTB = _B // _TB


def _tc_body(tt_ref, p0_ref, p1_ref, base_ref,
             w1c_ref, w2c_ref, wc_ref, o_ref):
    acc = p0_ref[...] + p1_ref[...]

    tt = tt_ref[...]
    w2c = w2c_ref[...]
    nte_l = [acc[:, l * _U:(l + 1) * _U] for l in range(_L)]
    att_s = []
    for l in range(_L):
        h = jnp.dot(nte_l[l], w1c_ref[...],
                    preferred_element_type=jnp.float32)
        g = jnp.tanh(h) * w2c
        sc = jnp.zeros((_TB, 1), jnp.float32)
        for t in range(_L):
            cs = jnp.sum(g[:, t * _U:(t + 1) * _U], axis=1, keepdims=True)
            sc = jnp.where(tt == t, cs, sc)
        att_s.append(sc)
    m = jnp.maximum(jnp.maximum(att_s[0], att_s[1]),
                    jnp.maximum(att_s[2], att_s[3]))
    e = [jnp.exp(x - m) for x in att_s]
    z = e[0] + e[1] + e[2] + e[3]
    nta = ((e[0] / z) * nte_l[0] + (e[1] / z) * nte_l[1]
           + (e[2] / z) * nte_l[2] + (e[3] / z) * nte_l[3])
    z_all = jnp.dot(nta, wc_ref[...],
                    preferred_element_type=jnp.float32)
    proj = jnp.zeros((_TB, _E), jnp.float32)
    for t in range(_L):
        proj = jnp.where(tt == t, z_all[:, t * _E:(t + 1) * _E], proj)
    pre = base_ref[...] + proj
    nrm = jnp.sqrt(jnp.sum(pre * pre, axis=1, keepdims=True))
    o_ref[...] = pre / jnp.maximum(nrm, 1e-12)


def _tc_stage(tt_col, part, base, w1c, w2c, wc):
    return pl.pallas_call(
        _tc_body,
        grid=(_NTB,),
        in_specs=[
            pl.BlockSpec((_TB, 1), lambda i: (i, 0)),
            pl.BlockSpec((_TB, _E), lambda i: (i, 0)),
            pl.BlockSpec((_TB, _E), lambda i: (i + _NTB, 0)),
            pl.BlockSpec((_TB, _E), lambda i: (i, 0)),
            pl.BlockSpec((_U, _L * _U), lambda i: (0, 0)),
            pl.BlockSpec((1, _L * _U), lambda i: (0, 0)),
            pl.BlockSpec((_U, _L * _E), lambda i: (0, 0)),
        ],
        out_specs=pl.BlockSpec((_TB, _E), lambda i: (i, 0)),
        out_shape=jax.ShapeDtypeStruct((_B, _E), jnp.float32),
        compiler_params=pltpu.CompilerParams(
            dimension_semantics=("arbitrary",),
        ),
    )(tt_col, part, part, base, w1c, w2c, wc)


def kernel(train_inputs, train_types, node_neigh, node_embeddings,
           node_type_embeddings, trans_weights, trans_weights_s1,
           trans_weights_s2):
    ti = train_inputs.astype(jnp.int32)
    tt = train_types.astype(jnp.int32)
    neigh_flat = node_neigh.astype(jnp.int32).reshape(-1)
    nte_flat = node_type_embeddings.reshape(_NN, _L * _U)
    s, base, owner = _sc_stage(neigh_flat, ti, nte_flat, node_embeddings)
    zeros = jnp.zeros((_TPS, _E), jnp.float32)
    part = _dedup_stage(ti, s, owner, zeros)
    w1c = jnp.concatenate([trans_weights_s1[t] for t in range(_L)], axis=1)
    w2c = trans_weights_s2[:, :, 0].reshape(1, _L * _U)
    wc = jnp.concatenate([trans_weights[t] for t in range(_L)], axis=1)
    return _tc_stage(tt.reshape(_B, 1), part, base, w1c, w2c, wc)


# speedup vs baseline: 1.4513x; 1.2072x over previous
import functools

import jax
import jax.numpy as jnp
from jax import lax
from jax.experimental import pallas as pl
from jax.experimental.pallas import tpu as pltpu
from jax.experimental.pallas import tpu_sc as plsc

_NN = 50000
_E = 128
_U = 32
_L = 4
_B = 4096
_NS = 10

_NW = 32
_RPW = _B // _NW
_BB = 8
_NBLK = _RPW // _BB
_GPB = _BB * _L * _NS
_GCS = (128, 128, 64)
_IDXN = _RPW * _L * _NS


def _sc_body(neigh_hbm, ti_hbm, nte_hbm, nemb_hbm,
             s_hbm, base_hbm, owner_hbm,
             neigh_v, buf0_v, buf1_v, sbuf_v, ti_v, bbuf_v, rid_v,
             sem0, sem1, bsem, osem):
    c = lax.axis_index("c")
    s = lax.axis_index("s")
    wid = s * 2 + c
    row0 = wid * _RPW

    pltpu.sync_copy(neigh_hbm.at[pl.ds(row0 * _L * _NS, _IDXN)], neigh_v)
    pltpu.sync_copy(ti_hbm.at[pl.ds(row0, _RPW)], ti_v)
    base_cp = pltpu.async_copy(nemb_hbm.at[ti_v], bbuf_v, bsem)

    for j in range(_RPW // 16):
        rid_v[pl.ds(j * 16, 16)] = lax.iota(jnp.int32, 16) + (row0 + j * 16)
    pltpu.async_copy(rid_v, owner_hbm.at[ti_v], osem).wait()

    bufs = (buf0_v, buf1_v)
    sems = (sem0, sem1)

    def fire(blk):
        gbase = blk * _GPB
        buf = bufs[blk % 2]
        sem = sems[blk % 2]
        cps = []
        off = 0
        for gc in _GCS:
            cps.append(pltpu.async_copy(
                nte_hbm.at[neigh_v.at[pl.ds(gbase + off, gc)]],
                buf.at[pl.ds(off, gc)],
                sem,
            ))
            off += gc
        return cps

    pend = fire(0)
    for blk in range(_NBLK):
        nxt = fire(blk + 1) if blk + 1 < _NBLK else []
        for cp in pend:
            cp.wait()
        pend = nxt
        buf = bufs[blk % 2]

        def acc_body(d, carry):
            r0 = d * _NS
            l = d % _L
            col = l * _U
            a0 = buf[r0, pl.ds(col, 16)]
            a1 = buf[r0, pl.ds(col + 16, 16)]
            for r in range(1, _NS):
                a0 = a0 + buf[r0 + r, pl.ds(col, 16)]
                a1 = a1 + buf[r0 + r, pl.ds(col + 16, 16)]
            b_loc = d // _L
            sbuf_v[blk * _BB + b_loc, pl.ds(col, 16)] = a0
            sbuf_v[blk * _BB + b_loc, pl.ds(col + 16, 16)] = a1
            return carry

        lax.fori_loop(0, _BB * _L, acc_body, 0)

    base_cp.wait()
    pltpu.sync_copy(sbuf_v, s_hbm.at[pl.ds(row0, _RPW)])
    pltpu.sync_copy(bbuf_v, base_hbm.at[pl.ds(row0, _RPW)])


def _sc_stage(neigh_flat, ti, nte_flat, nemb):
    f = pl.kernel(
        _sc_body,
        out_type=(
            jax.ShapeDtypeStruct((_B, _E), jnp.float32),
            jax.ShapeDtypeStruct((_B, _E), jnp.float32),
            jax.ShapeDtypeStruct((_NN,), jnp.int32),
        ),
        mesh=plsc.VectorSubcoreMesh(core_axis_name="c", subcore_axis_name="s"),
        scratch_types=[
            pltpu.VMEM((_IDXN,), jnp.int32),
            pltpu.VMEM((_GPB, _L * _U), jnp.float32),
            pltpu.VMEM((_GPB, _L * _U), jnp.float32),
            pltpu.VMEM((_RPW, _E), jnp.float32),
            pltpu.VMEM((_RPW,), jnp.int32),
            pltpu.VMEM((_RPW, _E), jnp.float32),
            pltpu.VMEM((_RPW,), jnp.int32),
            pltpu.SemaphoreType.DMA,
            pltpu.SemaphoreType.DMA,
            pltpu.SemaphoreType.DMA,
            pltpu.SemaphoreType.DMA,
        ],
    )
    return f(neigh_flat, ti, nte_flat, nemb)


_NSUB = 16
_TPS = _B // _NSUB


def _dedup_body(ti_hbm, s_hbm, owner_hbm, zeros_hbm, part_hbm,
                table_sh, ti_v, rep_v, sbuf_v, ti2_v, rep2_v, obuf_v, sem):
    c = lax.axis_index("c")
    s = lax.axis_index("s")
    wid = s * 2 + c
    row0 = wid * _RPW
    pltpu.sync_copy(ti_hbm.at[pl.ds(row0, _RPW)], ti_v)
    pltpu.async_copy(owner_hbm.at[ti_v], rep_v, sem).wait()
    pltpu.sync_copy(s_hbm.at[pl.ds(row0, _RPW)], sbuf_v)
    pltpu.sync_copy(zeros_hbm, table_sh.at[pl.ds(s * _TPS, _TPS)])
    plsc.subcore_barrier()
    pltpu.sync_copy(sbuf_v, table_sh.at[rep_v], add=True)
    plsc.subcore_barrier()
    g0 = s * _TPS
    pltpu.sync_copy(ti_hbm.at[pl.ds(g0, _TPS)], ti2_v)
    cps = [pltpu.async_copy(owner_hbm.at[ti2_v.at[pl.ds(k * 128, 128)]],
                            rep2_v.at[pl.ds(k * 128, 128)], sem)
           for k in range(_TPS // 128)]
    for cp in cps:
        cp.wait()
    cps = [pltpu.async_copy(table_sh.at[rep2_v.at[pl.ds(k * 128, 128)]],
                            obuf_v.at[pl.ds(k * 128, 128)], sem)
           for k in range(_TPS // 128)]
    for cp in cps:
        cp.wait()
    pltpu.sync_copy(obuf_v, part_hbm.at[pl.ds(c * _B + g0, _TPS)])


def _dedup_stage(ti, s, owner, zeros):
    f = pl.kernel(
        _dedup_body,
        out_type=jax.ShapeDtypeStruct((2 * _B, _E), jnp.float32),
        mesh=plsc.VectorSubcoreMesh(core_axis_name="c", subcore_axis_name="s"),
        scratch_types=[
            pltpu.VMEM_SHARED((_B, _E), jnp.float32),
            pltpu.VMEM((_RPW,), jnp.int32),
            pltpu.VMEM((_RPW,), jnp.int32),
            pltpu.VMEM((_RPW, _E), jnp.float32),
            pltpu.VMEM((_TPS,), jnp.int32),
            pltpu.VMEM((_TPS,), jnp.int32),
            pltpu.VMEM((_TPS, _E), jnp.float32),
            pltpu.SemaphoreType.DMA,
        ],
    )
    return f(ti, s, owner, zeros)


_TB = 256
_NTB = _B // _TB


def _tc_body(tt_ref, p0_ref, p1_ref, base_ref,
             w1big_ref, w2r_ref, wc_ref, o_ref):
    acc = p0_ref[...] + p1_ref[...]
    tt = tt_ref[...]
    nte_l = [acc[:, l * _U:(l + 1) * _U] for l in range(_L)]

    h = jnp.dot(acc, w1big_ref[...],
                preferred_element_type=jnp.float32)
    th = jnp.tanh(h)
    sr = jnp.dot(th, w2r_ref[...],
                 preferred_element_type=jnp.float32)
    srep = jnp.zeros((_TB, _E), jnp.float32)
    for t in range(_L):
        srep = jnp.where(tt == t, sr[:, t * _E:(t + 1) * _E], srep)
    sl = [srep[:, l * _U:(l + 1) * _U] for l in range(_L)]
    m = jnp.maximum(jnp.maximum(sl[0], sl[1]), jnp.maximum(sl[2], sl[3]))
    e = [jnp.exp(x - m) for x in sl]
    z = e[0] + e[1] + e[2] + e[3]
    nta = (e[0] * nte_l[0] + e[1] * nte_l[1]
           + e[2] * nte_l[2] + e[3] * nte_l[3]) / z
    z_all = jnp.dot(nta, wc_ref[...],
                    preferred_element_type=jnp.float32)
    proj = jnp.zeros((_TB, _E), jnp.float32)
    for t in range(_L):
        proj = jnp.where(tt == t, z_all[:, t * _E:(t + 1) * _E], proj)
    pre = base_ref[...] + proj
    nrm = jnp.sqrt(jnp.sum(pre * pre, axis=1, keepdims=True))
    o_ref[...] = pre / jnp.maximum(nrm, 1e-12)


def _tc_stage(tt_col, part, base, w1big, w2r, wc):
    return pl.pallas_call(
        _tc_body,
        grid=(_NTB,),
        in_specs=[
            pl.BlockSpec((_TB, 1), lambda i: (i, 0)),
            pl.BlockSpec((_TB, _E), lambda i: (i, 0)),
            pl.BlockSpec((_TB, _E), lambda i: (i + _NTB, 0)),
            pl.BlockSpec((_TB, _E), lambda i: (i, 0)),
            pl.BlockSpec((_E, _L * _E), lambda i: (0, 0)),
            pl.BlockSpec((_L * _E, _L * _E), lambda i: (0, 0)),
            pl.BlockSpec((_U, _L * _E), lambda i: (0, 0)),
        ],
        out_specs=pl.BlockSpec((_TB, _E), lambda i: (i, 0)),
        out_shape=jax.ShapeDtypeStruct((_B, _E), jnp.float32),
        compiler_params=pltpu.CompilerParams(
            dimension_semantics=("arbitrary",),
        ),
    )(tt_col, part, part, base, w1big, w2r, wc)


def kernel(train_inputs, train_types, node_neigh, node_embeddings,
           node_type_embeddings, trans_weights, trans_weights_s1,
           trans_weights_s2):
    ti = train_inputs.astype(jnp.int32)
    tt = train_types.astype(jnp.int32)
    neigh_flat = node_neigh.astype(jnp.int32).reshape(-1)
    nte_flat = node_type_embeddings.reshape(_NN, _L * _U)
    s, base, owner = _sc_stage(neigh_flat, ti, nte_flat, node_embeddings)
    zeros = jnp.zeros((_TPS, _E), jnp.float32)
    part = _dedup_stage(ti, s, owner, zeros)
    w1cat = jnp.concatenate([trans_weights_s1[t] for t in range(_L)], axis=1)
    w1big = jnp.zeros((_E, _L * _E), jnp.float32)
    for l in range(_L):
        w1big = w1big.at[l * _U:(l + 1) * _U, l * _E:(l + 1) * _E].set(w1cat)
    w2r = jnp.zeros((_L * _E, _L * _E), jnp.float32)
    for l in range(_L):
        for t in range(_L):
            blk = jnp.broadcast_to(trans_weights_s2[t, :, 0][:, None],
                                   (_U, _U))
            w2r = w2r.at[l * _E + t * _U:l * _E + (t + 1) * _U,
                         t * _E + l * _U:t * _E + (l + 1) * _U].set(blk)
    wc = jnp.concatenate([trans_weights[t] for t in range(_L)], axis=1)
    return _tc_stage(tt.reshape(_B, 1), part, base, w1big, w2r, wc)


# speedup vs baseline: 1.5009x; 1.0342x over previous
import functools

import jax
import jax.numpy as jnp
from jax import lax
from jax.experimental import pallas as pl
from jax.experimental.pallas import tpu as pltpu
from jax.experimental.pallas import tpu_sc as plsc

_NN = 50000
_E = 128
_U = 32
_L = 4
_B = 4096
_NS = 10

_NW = 32
_RPW = _B // _NW
_BB = 4
_NBLK = _RPW // _BB
_GPB = _BB * _L * _NS
_GCS = (128, 32)
_NBUF = 4
_IDXN = _RPW * _L * _NS


def _sc_body(neigh_hbm, ti_hbm, nte_hbm, nemb_hbm,
             s_hbm, base_hbm, owner_hbm,
             neigh_v, buf0_v, buf1_v, buf2_v, buf3_v, sbuf_v, ti_v, bbuf_v,
             rid_v, sem0, sem1, sem2, sem3, bsem, osem):
    c = lax.axis_index("c")
    s = lax.axis_index("s")
    wid = s * 2 + c
    row0 = wid * _RPW

    pltpu.sync_copy(neigh_hbm.at[pl.ds(row0 * _L * _NS, _IDXN)], neigh_v)
    pltpu.sync_copy(ti_hbm.at[pl.ds(row0, _RPW)], ti_v)
    base_cp = pltpu.async_copy(nemb_hbm.at[ti_v], bbuf_v, bsem)

    for j in range(_RPW // 16):
        rid_v[pl.ds(j * 16, 16)] = lax.iota(jnp.int32, 16) + (row0 + j * 16)
    pltpu.async_copy(rid_v, owner_hbm.at[ti_v], osem).wait()

    bufs = (buf0_v, buf1_v, buf2_v, buf3_v)
    sems = (sem0, sem1, sem2, sem3)

    def fire(blk):
        gbase = blk * _GPB
        buf = bufs[blk % _NBUF]
        sem = sems[blk % _NBUF]
        cps = []
        off = 0
        for gc in _GCS:
            cps.append(pltpu.async_copy(
                nte_hbm.at[neigh_v.at[pl.ds(gbase + off, gc)]],
                buf.at[pl.ds(off, gc)],
                sem,
            ))
            off += gc
        return cps

    pend = [fire(b) for b in range(_NBUF - 1)]
    for blk in range(_NBLK):
        nxt = blk + _NBUF - 1
        if nxt < _NBLK:
            pend.append(fire(nxt))
        for cp in pend.pop(0):
            cp.wait()
        buf = bufs[blk % _NBUF]

        def acc_body(d, carry):
            r0 = d * _NS
            l = d % _L
            col = l * _U
            a0 = buf[r0, pl.ds(col, 16)]
            a1 = buf[r0, pl.ds(col + 16, 16)]
            for r in range(1, _NS):
                a0 = a0 + buf[r0 + r, pl.ds(col, 16)]
                a1 = a1 + buf[r0 + r, pl.ds(col + 16, 16)]
            b_loc = d // _L
            sbuf_v[blk * _BB + b_loc, pl.ds(col, 16)] = a0
            sbuf_v[blk * _BB + b_loc, pl.ds(col + 16, 16)] = a1
            return carry

        lax.fori_loop(0, _BB * _L, acc_body, 0)

    base_cp.wait()
    pltpu.sync_copy(sbuf_v, s_hbm.at[pl.ds(row0, _RPW)])
    pltpu.sync_copy(bbuf_v, base_hbm.at[pl.ds(row0, _RPW)])


def _sc_stage(neigh_flat, ti, nte_flat, nemb):
    f = pl.kernel(
        _sc_body,
        out_type=(
            jax.ShapeDtypeStruct((_B, _E), jnp.float32),
            jax.ShapeDtypeStruct((_B, _E), jnp.float32),
            jax.ShapeDtypeStruct((_NN,), jnp.int32),
        ),
        mesh=plsc.VectorSubcoreMesh(core_axis_name="c", subcore_axis_name="s"),
        scratch_types=[
            pltpu.VMEM((_IDXN,), jnp.int32),
            pltpu.VMEM((_GPB, _L * _U), jnp.float32),
            pltpu.VMEM((_GPB, _L * _U), jnp.float32),
            pltpu.VMEM((_GPB, _L * _U), jnp.float32),
            pltpu.VMEM((_GPB, _L * _U), jnp.float32),
            pltpu.VMEM((_RPW, _E), jnp.float32),
            pltpu.VMEM((_RPW,), jnp.int32),
            pltpu.VMEM((_RPW, _E), jnp.float32),
            pltpu.VMEM((_RPW,), jnp.int32),
            pltpu.SemaphoreType.DMA,
            pltpu.SemaphoreType.DMA,
            pltpu.SemaphoreType.DMA,
            pltpu.SemaphoreType.DMA,
            pltpu.SemaphoreType.DMA,
            pltpu.SemaphoreType.DMA,
        ],
    )
    return f(neigh_flat, ti, nte_flat, nemb)


_NSUB = 16
_TPS = _B // _NSUB


def _dedup_body(ti_hbm, s_hbm, owner_hbm, zeros_hbm, part_hbm,
                table_sh, ti_v, rep_v, sbuf_v, ti2_v, rep2_v, obuf_v,
                sem, sem2, sem3):
    c = lax.axis_index("c")
    s = lax.axis_index("s")
    wid = s * 2 + c
    row0 = wid * _RPW
    g0 = s * _TPS
    pltpu.sync_copy(ti_hbm.at[pl.ds(row0, _RPW)], ti_v)
    rep_cp = pltpu.async_copy(owner_hbm.at[ti_v], rep_v, sem)
    pltpu.sync_copy(ti_hbm.at[pl.ds(g0, _TPS)], ti2_v)
    rep2_cps = [pltpu.async_copy(owner_hbm.at[ti2_v.at[pl.ds(k * 128, 128)]],
                                 rep2_v.at[pl.ds(k * 128, 128)], sem2)
                for k in range(_TPS // 128)]
    s_cp = pltpu.async_copy(s_hbm.at[pl.ds(row0, _RPW)], sbuf_v, sem3)
    pltpu.sync_copy(zeros_hbm, table_sh.at[pl.ds(s * _TPS, _TPS)])
    rep_cp.wait()
    s_cp.wait()
    plsc.subcore_barrier()
    pltpu.sync_copy(sbuf_v, table_sh.at[rep_v], add=True)
    plsc.subcore_barrier()
    for cp in rep2_cps:
        cp.wait()
    cps = [pltpu.async_copy(table_sh.at[rep2_v.at[pl.ds(k * 128, 128)]],
                            obuf_v.at[pl.ds(k * 128, 128)], sem)
           for k in range(_TPS // 128)]
    for cp in cps:
        cp.wait()
    pltpu.sync_copy(obuf_v, part_hbm.at[pl.ds(c * _B + g0, _TPS)])


def _dedup_stage(ti, s, owner, zeros):
    f = pl.kernel(
        _dedup_body,
        out_type=jax.ShapeDtypeStruct((2 * _B, _E), jnp.float32),
        mesh=plsc.VectorSubcoreMesh(core_axis_name="c", subcore_axis_name="s"),
        scratch_types=[
            pltpu.VMEM_SHARED((_B, _E), jnp.float32),
            pltpu.VMEM((_RPW,), jnp.int32),
            pltpu.VMEM((_RPW,), jnp.int32),
            pltpu.VMEM((_RPW, _E), jnp.float32),
            pltpu.VMEM((_TPS,), jnp.int32),
            pltpu.VMEM((_TPS,), jnp.int32),
            pltpu.VMEM((_TPS, _E), jnp.float32),
            pltpu.SemaphoreType.DMA,
            pltpu.SemaphoreType.DMA,
            pltpu.SemaphoreType.DMA,
        ],
    )
    return f(ti, s, owner, zeros)


_TB = 256
_NTB = _B // _TB


def _tc_body(tt_ref, p0_ref, p1_ref, base_ref,
             w1big_ref, w2r_ref, wc_ref, o_ref):
    acc = p0_ref[...] + p1_ref[...]
    tt = tt_ref[...]
    nte_l = [acc[:, l * _U:(l + 1) * _U] for l in range(_L)]

    h = jnp.dot(acc, w1big_ref[...],
                preferred_element_type=jnp.float32)
    th = jnp.tanh(h)
    sr = jnp.dot(th, w2r_ref[...],
                 preferred_element_type=jnp.float32)
    srep = jnp.zeros((_TB, _E), jnp.float32)
    for t in range(_L):
        srep = jnp.where(tt == t, sr[:, t * _E:(t + 1) * _E], srep)
    sl = [srep[:, l * _U:(l + 1) * _U] for l in range(_L)]
    m = jnp.maximum(jnp.maximum(sl[0], sl[1]), jnp.maximum(sl[2], sl[3]))
    e = [jnp.exp(x - m) for x in sl]
    z = e[0] + e[1] + e[2] + e[3]
    nta = (e[0] * nte_l[0] + e[1] * nte_l[1]
           + e[2] * nte_l[2] + e[3] * nte_l[3]) / z
    z_all = jnp.dot(nta, wc_ref[...],
                    preferred_element_type=jnp.float32)
    proj = jnp.zeros((_TB, _E), jnp.float32)
    for t in range(_L):
        proj = jnp.where(tt == t, z_all[:, t * _E:(t + 1) * _E], proj)
    pre = base_ref[...] + proj
    nrm = jnp.sqrt(jnp.sum(pre * pre, axis=1, keepdims=True))
    o_ref[...] = pre / jnp.maximum(nrm, 1e-12)


def _tc_stage(tt_col, part, base, w1big, w2r, wc):
    return pl.pallas_call(
        _tc_body,
        grid=(_NTB,),
        in_specs=[
            pl.BlockSpec((_TB, 1), lambda i: (i, 0)),
            pl.BlockSpec((_TB, _E), lambda i: (i, 0)),
            pl.BlockSpec((_TB, _E), lambda i: (i + _NTB, 0)),
            pl.BlockSpec((_TB, _E), lambda i: (i, 0)),
            pl.BlockSpec((_E, _L * _E), lambda i: (0, 0)),
            pl.BlockSpec((_L * _E, _L * _E), lambda i: (0, 0)),
            pl.BlockSpec((_U, _L * _E), lambda i: (0, 0)),
        ],
        out_specs=pl.BlockSpec((_TB, _E), lambda i: (i, 0)),
        out_shape=jax.ShapeDtypeStruct((_B, _E), jnp.float32),
        compiler_params=pltpu.CompilerParams(
            dimension_semantics=("arbitrary",),
        ),
    )(tt_col, part, part, base, w1big, w2r, wc)


def kernel(train_inputs, train_types, node_neigh, node_embeddings,
           node_type_embeddings, trans_weights, trans_weights_s1,
           trans_weights_s2):
    ti = train_inputs.astype(jnp.int32)
    tt = train_types.astype(jnp.int32)
    neigh_flat = node_neigh.astype(jnp.int32).reshape(-1)
    nte_flat = node_type_embeddings.reshape(_NN, _L * _U)
    s, base, owner = _sc_stage(neigh_flat, ti, nte_flat, node_embeddings)
    zeros = jnp.zeros((_TPS, _E), jnp.float32)
    part = _dedup_stage(ti, s, owner, zeros)
    w1cat = jnp.concatenate([trans_weights_s1[t] for t in range(_L)], axis=1)
    w1big = jnp.zeros((_E, _L * _E), jnp.float32)
    for l in range(_L):
        w1big = w1big.at[l * _U:(l + 1) * _U, l * _E:(l + 1) * _E].set(w1cat)
    w2r = jnp.zeros((_L * _E, _L * _E), jnp.float32)
    for l in range(_L):
        for t in range(_L):
            blk = jnp.broadcast_to(trans_weights_s2[t, :, 0][:, None],
                                   (_U, _U))
            w2r = w2r.at[l * _E + t * _U:l * _E + (t + 1) * _U,
                         t * _E + l * _U:t * _E + (l + 1) * _U].set(blk)
    wc = jnp.concatenate([trans_weights[t] for t in range(_L)], axis=1)
    return _tc_stage(tt.reshape(_B, 1), part, base, w1big, w2r, wc)


# speedup vs baseline: 1.6482x; 1.0982x over previous
import functools

import jax
import jax.numpy as jnp
from jax import lax
from jax.experimental import pallas as pl
from jax.experimental.pallas import tpu as pltpu
from jax.experimental.pallas import tpu_sc as plsc

_NN = 50000
_E = 128
_U = 32
_L = 4
_B = 4096
_NS = 10

_NW = 32
_RPW = _B // _NW
_BB = 4
_NBLK = _RPW // _BB
_GPB = _BB * _L * _NS
_GCS = (128, 32)
_NBUF = 4
_IDXN = _RPW * _L * _NS


def _sc_body(neigh_hbm, ti_hbm, nte_hbm, nemb_hbm,
             s_hbm, base_hbm, owner_hbm,
             neigh_v, buf0_v, buf1_v, buf2_v, buf3_v, sbuf_v, ti_v, bbuf_v,
             rid_v, sem0, sem1, sem2, sem3, bsem, osem):
    c = lax.axis_index("c")
    s = lax.axis_index("s")
    wid = s * 2 + c
    row0 = wid * _RPW

    pltpu.sync_copy(neigh_hbm.at[pl.ds(row0 * _L * _NS, _IDXN)], neigh_v)
    pltpu.sync_copy(ti_hbm.at[pl.ds(row0, _RPW)], ti_v)
    base_cp = pltpu.async_copy(nemb_hbm.at[ti_v], bbuf_v, bsem)

    for j in range(_RPW // 16):
        rid_v[pl.ds(j * 16, 16)] = lax.iota(jnp.int32, 16) + (row0 + j * 16)
    pltpu.async_copy(rid_v, owner_hbm.at[ti_v], osem).wait()

    bufs = (buf0_v, buf1_v, buf2_v, buf3_v)
    sems = (sem0, sem1, sem2, sem3)

    def fire(blk):
        gbase = blk * _GPB
        buf = bufs[blk % _NBUF]
        sem = sems[blk % _NBUF]
        cps = []
        off = 0
        for gc in _GCS:
            cps.append(pltpu.async_copy(
                nte_hbm.at[neigh_v.at[pl.ds(gbase + off, gc)]],
                buf.at[pl.ds(off, gc)],
                sem,
            ))
            off += gc
        return cps

    pend = [fire(b) for b in range(_NBUF - 1)]
    for blk in range(_NBLK):
        nxt = blk + _NBUF - 1
        if nxt < _NBLK:
            pend.append(fire(nxt))
        for cp in pend.pop(0):
            cp.wait()
        buf = bufs[blk % _NBUF]

        def acc_body(d, carry):
            r0 = d * _NS
            l = d % _L
            col = l * _U
            a0 = buf[r0, pl.ds(col, 16)]
            a1 = buf[r0, pl.ds(col + 16, 16)]
            for r in range(1, _NS):
                a0 = a0 + buf[r0 + r, pl.ds(col, 16)]
                a1 = a1 + buf[r0 + r, pl.ds(col + 16, 16)]
            b_loc = d // _L
            sbuf_v[blk * _BB + b_loc, pl.ds(col, 16)] = a0
            sbuf_v[blk * _BB + b_loc, pl.ds(col + 16, 16)] = a1
            return carry

        lax.fori_loop(0, _BB * _L, acc_body, 0)

    base_cp.wait()
    pltpu.sync_copy(sbuf_v, s_hbm.at[pl.ds(row0, _RPW)])
    pltpu.sync_copy(bbuf_v, base_hbm.at[pl.ds(row0, _RPW)])


def _sc_stage(neigh_flat, ti, nte_flat, nemb):
    f = pl.kernel(
        _sc_body,
        out_type=(
            jax.ShapeDtypeStruct((_B, _E), jnp.float32),
            jax.ShapeDtypeStruct((_B, _E), jnp.float32),
            jax.ShapeDtypeStruct((_NN,), jnp.int32),
        ),
        mesh=plsc.VectorSubcoreMesh(core_axis_name="c", subcore_axis_name="s"),
        scratch_types=[
            pltpu.VMEM((_IDXN,), jnp.int32),
            pltpu.VMEM((_GPB, _L * _U), jnp.float32),
            pltpu.VMEM((_GPB, _L * _U), jnp.float32),
            pltpu.VMEM((_GPB, _L * _U), jnp.float32),
            pltpu.VMEM((_GPB, _L * _U), jnp.float32),
            pltpu.VMEM((_RPW, _E), jnp.float32),
            pltpu.VMEM((_RPW,), jnp.int32),
            pltpu.VMEM((_RPW, _E), jnp.float32),
            pltpu.VMEM((_RPW,), jnp.int32),
            pltpu.SemaphoreType.DMA,
            pltpu.SemaphoreType.DMA,
            pltpu.SemaphoreType.DMA,
            pltpu.SemaphoreType.DMA,
            pltpu.SemaphoreType.DMA,
            pltpu.SemaphoreType.DMA,
        ],
    )
    return f(neigh_flat, ti, nte_flat, nemb)


_NSUB = 16
_TPS = _B // _NSUB


def _dedup_body(ti_hbm, s_hbm, owner_hbm, zeros_hbm, part_hbm,
                table_sh, ti_v, rep_v, sbuf_v, ti2_v, rep2_v, obuf_v,
                sem, sem2, sem3):
    c = lax.axis_index("c")
    s = lax.axis_index("s")
    wid = s * 2 + c
    row0 = wid * _RPW
    g0 = s * _TPS
    pltpu.sync_copy(ti_hbm.at[pl.ds(row0, _RPW)], ti_v)
    rep_cp = pltpu.async_copy(owner_hbm.at[ti_v], rep_v, sem)
    pltpu.sync_copy(ti_hbm.at[pl.ds(g0, _TPS)], ti2_v)
    rep2_cps = [pltpu.async_copy(owner_hbm.at[ti2_v.at[pl.ds(k * 128, 128)]],
                                 rep2_v.at[pl.ds(k * 128, 128)], sem2)
                for k in range(_TPS // 128)]
    s_cp = pltpu.async_copy(s_hbm.at[pl.ds(row0, _RPW)], sbuf_v, sem3)
    pltpu.sync_copy(zeros_hbm, table_sh.at[pl.ds(s * _TPS, _TPS)])
    rep_cp.wait()
    s_cp.wait()
    plsc.subcore_barrier()
    pltpu.sync_copy(sbuf_v, table_sh.at[rep_v], add=True)
    plsc.subcore_barrier()
    for cp in rep2_cps:
        cp.wait()
    cps = [pltpu.async_copy(table_sh.at[rep2_v.at[pl.ds(k * 128, 128)]],
                            obuf_v.at[pl.ds(k * 128, 128)], sem)
           for k in range(_TPS // 128)]
    for cp in cps:
        cp.wait()
    pltpu.sync_copy(obuf_v, part_hbm.at[pl.ds(c * _B + g0, _TPS)])


def _dedup_stage(ti, s, owner, zeros):
    f = pl.kernel(
        _dedup_body,
        out_type=jax.ShapeDtypeStruct((2 * _B, _E), jnp.float32),
        mesh=plsc.VectorSubcoreMesh(core_axis_name="c", subcore_axis_name="s"),
        scratch_types=[
            pltpu.VMEM_SHARED((_B, _E), jnp.float32),
            pltpu.VMEM((_RPW,), jnp.int32),
            pltpu.VMEM((_RPW,), jnp.int32),
            pltpu.VMEM((_RPW, _E), jnp.float32),
            pltpu.VMEM((_TPS,), jnp.int32),
            pltpu.VMEM((_TPS,), jnp.int32),
            pltpu.VMEM((_TPS, _E), jnp.float32),
            pltpu.SemaphoreType.DMA,
            pltpu.SemaphoreType.DMA,
            pltpu.SemaphoreType.DMA,
        ],
    )
    return f(ti, s, owner, zeros)


_TB = 512
_NTB = _B // _TB


def _tc_body(tt_ref, p0_ref, p1_ref, base_ref,
             w1big_ref, w2r_ref, wc_ref, o_ref):
    acc = p0_ref[...] + p1_ref[...]
    tt = tt_ref[...]
    nte_l = [acc[:, l * _U:(l + 1) * _U] for l in range(_L)]

    h = jnp.dot(acc, w1big_ref[...],
                preferred_element_type=jnp.float32)
    th = jnp.tanh(h)
    sr = jnp.dot(th, w2r_ref[...],
                 preferred_element_type=jnp.float32)
    srep = jnp.zeros((_TB, _E), jnp.float32)
    for t in range(_L):
        srep = jnp.where(tt == t, sr[:, t * _E:(t + 1) * _E], srep)
    sl = [srep[:, l * _U:(l + 1) * _U] for l in range(_L)]
    m = jnp.maximum(jnp.maximum(sl[0], sl[1]), jnp.maximum(sl[2], sl[3]))
    e = [jnp.exp(x - m) for x in sl]
    z = e[0] + e[1] + e[2] + e[3]
    nta = (e[0] * nte_l[0] + e[1] * nte_l[1]
           + e[2] * nte_l[2] + e[3] * nte_l[3]) / z
    z_all = jnp.dot(nta, wc_ref[...],
                    preferred_element_type=jnp.float32)
    proj = jnp.zeros((_TB, _E), jnp.float32)
    for t in range(_L):
        proj = jnp.where(tt == t, z_all[:, t * _E:(t + 1) * _E], proj)
    pre = base_ref[...] + proj
    nrm = jnp.sqrt(jnp.sum(pre * pre, axis=1, keepdims=True))
    o_ref[...] = pre / jnp.maximum(nrm, 1e-12)


def _tc_stage(tt_col, part, base, w1big, w2r, wc):
    return pl.pallas_call(
        _tc_body,
        grid=(_NTB,),
        in_specs=[
            pl.BlockSpec((_TB, 1), lambda i: (i, 0)),
            pl.BlockSpec((_TB, _E), lambda i: (i, 0)),
            pl.BlockSpec((_TB, _E), lambda i: (i + _NTB, 0)),
            pl.BlockSpec((_TB, _E), lambda i: (i, 0)),
            pl.BlockSpec((_E, _L * _E), lambda i: (0, 0)),
            pl.BlockSpec((_L * _E, _L * _E), lambda i: (0, 0)),
            pl.BlockSpec((_U, _L * _E), lambda i: (0, 0)),
        ],
        out_specs=pl.BlockSpec((_TB, _E), lambda i: (i, 0)),
        out_shape=jax.ShapeDtypeStruct((_B, _E), jnp.float32),
        compiler_params=pltpu.CompilerParams(
            dimension_semantics=("arbitrary",),
        ),
    )(tt_col, part, part, base, w1big, w2r, wc)


def kernel(train_inputs, train_types, node_neigh, node_embeddings,
           node_type_embeddings, trans_weights, trans_weights_s1,
           trans_weights_s2):
    ti = train_inputs.astype(jnp.int32)
    tt = train_types.astype(jnp.int32)
    neigh_flat = node_neigh.astype(jnp.int32).reshape(-1)
    nte_flat = node_type_embeddings.reshape(_NN, _L * _U)
    s, base, owner = _sc_stage(neigh_flat, ti, nte_flat, node_embeddings)
    zeros = jnp.zeros((_TPS, _E), jnp.float32)
    part = _dedup_stage(ti, s, owner, zeros)
    eye = jnp.eye(_L, dtype=jnp.float32)
    w1cat = trans_weights_s1.transpose(1, 0, 2).reshape(_U, _L * _U)
    w1big = jnp.kron(eye, w1cat)
    w2r = jnp.einsum("lm,tn,ta,u->ltanmu", eye, eye,
                     trans_weights_s2[:, :, 0],
                     jnp.ones((_U,), jnp.float32)).reshape(_L * _E, _L * _E)
    wc = trans_weights.transpose(1, 0, 2).reshape(_U, _L * _E)
    return _tc_stage(tt.reshape(_B, 1), part, base, w1big, w2r, wc)
